# baseline (speedup 1.0000x reference)
import functools

import jax
import jax.numpy as jnp
from jax import lax
from jax.experimental import pallas as pl
from jax.experimental.pallas import tpu as pltpu
from jax.experimental.pallas import tpu_sc as plsc

GRID_N = 256
NPTS = 8192 * 128
NW = 32
PER_W = NPTS // NW
CHUNK = 128
NCHUNK = PER_W // CHUNK
L = 16
MAGIC = 12582912.0


def _pack_body(mask_ref, bound_ref, out_ref):
    c = jnp.logical_and(mask_ref[...], bound_ref[...])
    c = c.reshape(8, 8, 32, 256)
    b = lax.broadcasted_iota(jnp.int32, (1, 1, 32, 1), 2)
    bits = jnp.where(c, jnp.left_shift(jnp.int32(1), b), 0)
    out_ref[...] = jnp.sum(bits, axis=2)


def _pack(mask, bound_mask):
    return pl.pallas_call(
        _pack_body,
        grid=(GRID_N // 8,),
        in_specs=[
            pl.BlockSpec((8, GRID_N, GRID_N), lambda i: (i, 0, 0)),
            pl.BlockSpec((8, GRID_N, GRID_N), lambda i: (i, 0, 0)),
        ],
        out_specs=pl.BlockSpec((8, 8, GRID_N), lambda i: (i, 0, 0)),
        out_shape=jax.ShapeDtypeStruct((GRID_N, 8, GRID_N), jnp.int32),
    )(mask, bound_mask)


def _sc_body(xyz_hbm, table_hbm, params_hbm, out_hbm,
             pv, xv, wv, av, gv, ov, sem):
    wid = lax.axis_index("s") * 2 + lax.axis_index("c")
    base_pt = wid * PER_W

    pltpu.sync_copy(params_hbm, pv)
    sx = pv[pl.ds(0, L)]
    sy = pv[pl.ds(L, L)]
    sz = pv[pl.ds(2 * L, L)]
    tx = pv[pl.ds(3 * L, L)]
    ty = pv[pl.ds(4 * L, L)]
    tz = pv[pl.ds(5 * L, L)]

    lane = lax.broadcasted_iota(jnp.int32, (L,), 0)

    def chunk_body(t, carry):
        pt0 = base_pt + t * CHUNK
        pltpu.sync_copy(xyz_hbm.at[pl.ds(pt0 * 3, CHUNK * 3)], xv)

        for g in range(CHUNK // L):
            ix = lane * 3 + (g * 3 * L)
            xs = plsc.load_gather(xv, [ix])
            ys = plsc.load_gather(xv, [ix + 1])
            zs = plsc.load_gather(xv, [ix + 2])

            fx = (xs * sx + tx + MAGIC) - MAGIC
            fy = (ys * sy + ty + MAGIC) - MAGIC
            fz = (zs * sz + tz + MAGIC) - MAGIC

            inb = ((fx >= 0.0) & (fx <= 255.0)
                   & (fy >= 0.0) & (fy <= 255.0)
                   & (fz >= 0.0) & (fz <= 255.0))

            ii = jnp.clip(fx.astype(jnp.int32), 0, 255)
            jj = jnp.clip(fy.astype(jnp.int32), 0, 255)
            kk = jnp.clip(fz.astype(jnp.int32), 0, 255)

            word = ii * 2048 + jnp.left_shift(
                lax.shift_right_logical(jj, 5), 8) + kk
            aux = (jj & 31) | jnp.where(inb, 32, 0)

            wv[pl.ds(g * L, L)] = word
            av[pl.ds(g * L, L)] = aux

        pltpu.async_copy(table_hbm.at[wv], gv, sem).wait()

        for g in range(CHUNK // L):
            w = gv[pl.ds(g * L, L)]
            a = av[pl.ds(g * L, L)]
            bit = a & 31
            val = (lax.shift_right_logical(w, bit) & 1) & lax.shift_right_logical(a, 5)
            ov[pl.ds(g * L, L)] = val

        pltpu.sync_copy(ov, out_hbm.at[pl.ds(pt0, CHUNK)])
        return carry

    lax.fori_loop(0, NCHUNK, chunk_body, 0)


@functools.partial(
    pl.kernel,
    out_type=jax.ShapeDtypeStruct((NPTS,), jnp.int32),
    mesh=plsc.VectorSubcoreMesh(core_axis_name="c", subcore_axis_name="s"),
    compiler_params=pltpu.CompilerParams(needs_layout_passes=False),
    scratch_types=[
        pltpu.VMEM((6 * L,), jnp.float32),
        pltpu.VMEM((CHUNK * 3,), jnp.float32),
        pltpu.VMEM((CHUNK,), jnp.int32),
        pltpu.VMEM((CHUNK,), jnp.int32),
        pltpu.VMEM((CHUNK,), jnp.int32),
        pltpu.VMEM((CHUNK,), jnp.int32),
        pltpu.SemaphoreType.DMA,
    ],
)
def _lookup(xyz_hbm, table_hbm, params_hbm, out_hbm, *scratch):
    _sc_body(xyz_hbm, table_hbm, params_hbm, out_hbm, *scratch)


def kernel(xyz, mask, bound_mask, xyz2ijk_scale, xyz2ijk_shift):
    shape = xyz.shape[:-1]
    packed = _pack(mask, bound_mask).reshape(-1)
    params = jnp.concatenate([
        jnp.repeat(xyz2ijk_scale.astype(jnp.float32), L),
        jnp.repeat(xyz2ijk_shift.astype(jnp.float32), L),
    ])
    flat = _lookup(xyz.reshape(-1), packed, params)
    return flat.astype(jnp.bool_).reshape(shape)


# speedup vs baseline: 1.4505x; 1.4505x over previous
import functools

import jax
import jax.numpy as jnp
from jax import lax
from jax.experimental import pallas as pl
from jax.experimental.pallas import tpu as pltpu
from jax.experimental.pallas import tpu_sc as plsc

GRID_N = 256
NPTS = 8192 * 128
NW = 32
PER_W = NPTS // NW
BC = 1024
NB = PER_W // BC
ROW = 128
NR = BC // ROW
L = 16
MAGIC = 12582912.0


def _pack_body(comb_ref, out_ref):
    c = comb_ref[...].astype(jnp.int32)
    c = c.reshape(8, 8, 32, 256)
    b = lax.broadcasted_iota(jnp.int32, (1, 1, 32, 1), 2)
    out_ref[...] = jnp.sum(c << b, axis=2)


def _pack(comb_u8):
    return pl.pallas_call(
        _pack_body,
        grid=(GRID_N // 8,),
        in_specs=[
            pl.BlockSpec((8, GRID_N, GRID_N), lambda i: (i, 0, 0)),
        ],
        out_specs=pl.BlockSpec((8, 8, GRID_N), lambda i: (i, 0, 0)),
        out_shape=jax.ShapeDtypeStruct((GRID_N, 8, GRID_N), jnp.int32),
    )(comb_u8)


def _sc_body(xyz_hbm, table_hbm, params_hbm, out_hbm,
             pv, xall, wv, av, gv, ov, insem, gsem):
    wid = lax.axis_index("s") * 2 + lax.axis_index("c")
    base_pt = wid * PER_W

    pltpu.sync_copy(params_hbm, pv)
    sx = pv[pl.ds(0, L)]
    sy = pv[pl.ds(L, L)]
    sz = pv[pl.ds(2 * L, L)]
    tx = pv[pl.ds(3 * L, L)]
    ty = pv[pl.ds(4 * L, L)]
    tz = pv[pl.ds(5 * L, L)]

    for c in range(3):
        pltpu.async_copy(xyz_hbm.at[pl.ds(c * NPTS + base_pt, BC)],
                         xall.at[pl.ds(c * BC, BC)], insem)

    def block_body(t, carry):
        pt0 = base_pt + t * BC
        pltpu.make_async_copy(xyz_hbm.at[pl.ds(0, 3 * BC)], xall, insem).wait()

        for r in range(NR):
            for gg in range(ROW // L):
                off = r * ROW + gg * L
                xs = xall[pl.ds(off, L)]
                ys = xall[pl.ds(BC + off, L)]
                zs = xall[pl.ds(2 * BC + off, L)]

                fx = (xs * sx + tx + MAGIC) - MAGIC
                fy = (ys * sy + ty + MAGIC) - MAGIC
                fz = (zs * sz + tz + MAGIC) - MAGIC

                inb = ((fx >= 0.0) & (fx <= 255.0)
                       & (fy >= 0.0) & (fy <= 255.0)
                       & (fz >= 0.0) & (fz <= 255.0))

                ii = fx.astype(jnp.int32)
                jj = fy.astype(jnp.int32)
                kk = fz.astype(jnp.int32)

                word = ii * 2048 + jnp.left_shift(
                    lax.shift_right_logical(jj, 5) & 7, 8) + kk
                wv[pl.ds(off, L)] = jnp.where(inb, word, 0)
                av[pl.ds(off, L)] = (jj & 31) | jnp.where(inb, 32, 0)

            pltpu.async_copy(table_hbm.at[wv.at[pl.ds(r * ROW, ROW)]],
                             gv.at[pl.ds(r * ROW, ROW)], gsem)

        ptn = base_pt + lax.rem(t + 1, NB) * BC
        for c in range(3):
            pltpu.async_copy(xyz_hbm.at[pl.ds(c * NPTS + ptn, BC)],
                             xall.at[pl.ds(c * BC, BC)], insem)

        pltpu.make_async_copy(table_hbm.at[pl.ds(0, BC)], gv, gsem).wait()

        for g in range(BC // L):
            w = gv[pl.ds(g * L, L)]
            a = av[pl.ds(g * L, L)]
            val = (lax.shift_right_logical(w, a & 31) & 1) \
                & lax.shift_right_logical(a, 5)
            ov[pl.ds(g * L, L)] = val

        pltpu.sync_copy(ov, out_hbm.at[pl.ds(pt0, BC)])
        return carry

    lax.fori_loop(0, NB, block_body, 0)
    pltpu.make_async_copy(xyz_hbm.at[pl.ds(0, 3 * BC)], xall, insem).wait()


@functools.partial(
    pl.kernel,
    out_type=jax.ShapeDtypeStruct((NPTS,), jnp.int32),
    mesh=plsc.VectorSubcoreMesh(core_axis_name="c", subcore_axis_name="s"),
    compiler_params=pltpu.CompilerParams(needs_layout_passes=False),
    scratch_types=[
        pltpu.VMEM((6 * L,), jnp.float32),
        pltpu.VMEM((3 * BC,), jnp.float32),
        pltpu.VMEM((BC,), jnp.int32),
        pltpu.VMEM((BC,), jnp.int32),
        pltpu.VMEM((BC,), jnp.int32),
        pltpu.VMEM((BC,), jnp.int32),
        pltpu.SemaphoreType.DMA,
        pltpu.SemaphoreType.DMA,
    ],
)
def _lookup(xyz_hbm, table_hbm, params_hbm, out_hbm, *scratch):
    _sc_body(xyz_hbm, table_hbm, params_hbm, out_hbm, *scratch)


def kernel(xyz, mask, bound_mask, xyz2ijk_scale, xyz2ijk_shift):
    shape = xyz.shape[:-1]
    comb_u8 = jnp.logical_and(mask, bound_mask).astype(jnp.uint8)
    packed = _pack(comb_u8).reshape(-1)
    xflat = jnp.transpose(xyz, (2, 0, 1)).reshape(-1)
    params = jnp.concatenate([
        jnp.repeat(xyz2ijk_scale.astype(jnp.float32), L),
        jnp.repeat(xyz2ijk_shift.astype(jnp.float32), L),
    ])
    flat = _lookup(xflat, packed, params)
    return flat.astype(jnp.bool_).reshape(shape)


# speedup vs baseline: 15.5858x; 10.7449x over previous
import functools

import jax
import jax.numpy as jnp
from jax import lax
from jax.experimental import pallas as pl
from jax.experimental.pallas import tpu as pltpu
from jax.experimental.pallas import tpu_sc as plsc

GRID_N = 256
NPTS = 8192 * 128
NW = 32
PER_W = NPTS // NW
BC = 1024
NB = PER_W // BC
ROW = 128
NR = BC // ROW
L = 16
TABLE_W = GRID_N * 8 * GRID_N
MAGIC = 12582912.0


def _pack_body(comb_ref, out_ref):
    c = comb_ref[...].astype(jnp.int32)
    c = c.reshape(8, 8, 32, 256)
    b = lax.broadcasted_iota(jnp.int32, (1, 1, 32, 1), 2)
    out_ref[...] = jnp.sum(c << b, axis=2)


def _pack(comb_u8):
    return pl.pallas_call(
        _pack_body,
        grid=(GRID_N // 8,),
        in_specs=[
            pl.BlockSpec((8, GRID_N, GRID_N), lambda i: (i, 0, 0)),
        ],
        out_specs=pl.BlockSpec((8, 8, GRID_N), lambda i: (i, 0, 0)),
        out_shape=jax.ShapeDtypeStruct((GRID_N, 8, GRID_N), jnp.int32),
    )(comb_u8)


def _sc_body(xyz_hbm, table_hbm, params_hbm, out_hbm,
             pv, xall, wv, av, gv, ov, tsh, insem, gsem):
    sid = lax.axis_index("s")
    wid = sid * 2 + lax.axis_index("c")
    base_pt = wid * PER_W

    TW = TABLE_W // 16
    pltpu.sync_copy(table_hbm.at[pl.ds(sid * TW, TW)],
                    tsh.at[pl.ds(sid * TW, TW)])
    plsc.subcore_barrier()

    pltpu.sync_copy(params_hbm, pv)
    sx = pv[pl.ds(0, L)]
    sy = pv[pl.ds(L, L)]
    sz = pv[pl.ds(2 * L, L)]
    tx = pv[pl.ds(3 * L, L)]
    ty = pv[pl.ds(4 * L, L)]
    tz = pv[pl.ds(5 * L, L)]

    for c in range(3):
        pltpu.async_copy(xyz_hbm.at[pl.ds(c * NPTS + base_pt, BC)],
                         xall.at[pl.ds(c * BC, BC)], insem)

    def block_body(t, carry):
        pt0 = base_pt + t * BC
        pltpu.make_async_copy(xyz_hbm.at[pl.ds(0, 3 * BC)], xall, insem).wait()

        for r in range(NR):
            for gg in range(ROW // L):
                off = r * ROW + gg * L
                xs = xall[pl.ds(off, L)]
                ys = xall[pl.ds(BC + off, L)]
                zs = xall[pl.ds(2 * BC + off, L)]

                fx = (xs * sx + tx + MAGIC) - MAGIC
                fy = (ys * sy + ty + MAGIC) - MAGIC
                fz = (zs * sz + tz + MAGIC) - MAGIC

                inb = ((fx >= 0.0) & (fx <= 255.0)
                       & (fy >= 0.0) & (fy <= 255.0)
                       & (fz >= 0.0) & (fz <= 255.0))

                ii = jnp.clip(fx.astype(jnp.int32), 0, 255)
                jj = jnp.clip(fy.astype(jnp.int32), 0, 255)
                kk = jnp.clip(fz.astype(jnp.int32), 0, 255)

                wv[pl.ds(off, L)] = ii * 2048 + jnp.left_shift(
                    lax.shift_right_logical(jj, 5), 8) + kk
                av[pl.ds(off, L)] = (jj & 31) | jnp.where(inb, 32, 0)

            pltpu.async_copy(tsh.at[wv.at[pl.ds(r * ROW, ROW)]],
                             gv.at[pl.ds(r * ROW, ROW)], gsem)

        ptn = base_pt + lax.rem(t + 1, NB) * BC
        for c in range(3):
            pltpu.async_copy(xyz_hbm.at[pl.ds(c * NPTS + ptn, BC)],
                             xall.at[pl.ds(c * BC, BC)], insem)

        pltpu.make_async_copy(table_hbm.at[pl.ds(0, BC)], gv, gsem).wait()

        for g in range(BC // L):
            w = gv[pl.ds(g * L, L)]
            a = av[pl.ds(g * L, L)]
            val = (lax.shift_right_logical(w, a & 31) & 1) \
                & lax.shift_right_logical(a, 5)
            ov[pl.ds(g * L, L)] = val

        pltpu.sync_copy(ov, out_hbm.at[pl.ds(pt0, BC)])
        return carry

    lax.fori_loop(0, NB, block_body, 0)
    pltpu.make_async_copy(xyz_hbm.at[pl.ds(0, 3 * BC)], xall, insem).wait()


@functools.partial(
    pl.kernel,
    out_type=jax.ShapeDtypeStruct((NPTS,), jnp.int32),
    mesh=plsc.VectorSubcoreMesh(core_axis_name="c", subcore_axis_name="s"),
    compiler_params=pltpu.CompilerParams(needs_layout_passes=False),
    scratch_types=[
        pltpu.VMEM((6 * L,), jnp.float32),
        pltpu.VMEM((3 * BC,), jnp.float32),
        pltpu.VMEM((BC,), jnp.int32),
        pltpu.VMEM((BC,), jnp.int32),
        pltpu.VMEM((BC,), jnp.int32),
        pltpu.VMEM((BC,), jnp.int32),
        pltpu.VMEM_SHARED((TABLE_W,), jnp.int32),
        pltpu.SemaphoreType.DMA,
        pltpu.SemaphoreType.DMA,
    ],
)
def _lookup(xyz_hbm, table_hbm, params_hbm, out_hbm, *scratch):
    _sc_body(xyz_hbm, table_hbm, params_hbm, out_hbm, *scratch)


def kernel(xyz, mask, bound_mask, xyz2ijk_scale, xyz2ijk_shift):
    shape = xyz.shape[:-1]
    comb_u8 = jnp.logical_and(mask, bound_mask).astype(jnp.uint8)
    packed = _pack(comb_u8).reshape(-1)
    xflat = jnp.transpose(xyz, (2, 0, 1)).reshape(-1)
    params = jnp.concatenate([
        jnp.repeat(xyz2ijk_scale.astype(jnp.float32), L),
        jnp.repeat(xyz2ijk_shift.astype(jnp.float32), L),
    ])
    flat = _lookup(xflat, packed, params)
    return flat.astype(jnp.bool_).reshape(shape)
